# Initial kernel scaffold; baseline (speedup 1.0000x reference)
import jax
import jax.numpy as jnp
from jax.experimental import pallas as pl


def kernel(x, edge_index, W1, b1, W2, b2, W3, b3):
    raise NotImplementedError("write your pallas kernel here")


# speedup vs baseline: 6.1983x; 6.1983x over previous
import jax
import jax.numpy as jnp
from jax import lax
from jax.experimental import pallas as pl
from jax.experimental.pallas import tpu as pltpu
from jax.experimental.pallas import tpu_sc as plsc

_NC = 2
_NS = 16
_NW = _NC * _NS
_CH = 80
_ZR = 125


_WIN = 25


def _make_sc_conv(n, d, e):
    ew = e // _NW
    nchunk = ew // _CH
    nwin = nchunk // _WIN
    _NT = 10
    rpt = n // _NT
    assert ew * _NW == e and nchunk * _CH == ew and nwin * _WIN == nchunk
    assert rpt % 8 == 0 and _NT * rpt == n

    mesh = plsc.VectorSubcoreMesh(core_axis_name="c", subcore_axis_name="s")

    def body(h_hbm, src_hbm, dst_hbm, z_hbm, out_hbm,
             src_v, dst_v, rows_v, acc_sh):
        c = lax.axis_index("c")
        s = lax.axis_index("s")
        w = s * _NC + c
        base = s * rpt

        @pl.when(s < _NT)
        def _zero_acc():
            pltpu.sync_copy(z_hbm.at[pl.ds(base, rpt)],
                            acc_sh.at[pl.ds(base, rpt)])

        plsc.subcore_barrier()

        def _win(wi, carry):
            pltpu.sync_copy(src_hbm.at[w * nwin + wi], src_v)
            pltpu.sync_copy(dst_hbm.at[w * nwin + wi], dst_v)

            def _edge(j, carry2):
                pltpu.sync_copy(h_hbm.at[src_v.at[j]], rows_v)
                pltpu.sync_copy(rows_v, acc_sh.at[dst_v.at[j]], add=True)
                return carry2
            lax.fori_loop(0, _WIN, _edge, 0)
            return carry
        lax.fori_loop(0, nwin, _win, 0)

        plsc.subcore_barrier()

        @pl.when(s < _NT)
        def _readout():
            pltpu.sync_copy(acc_sh.at[pl.ds(base, rpt)],
                            out_hbm.at[pl.ds(c * n + base, rpt)])

    return pl.kernel(
        body, out_type=jax.ShapeDtypeStruct((_NC * n, d), jnp.float32),
        mesh=mesh,
        scratch_types=[
            pltpu.VMEM((_WIN, _CH), jnp.int32),
            pltpu.VMEM((_WIN, _CH), jnp.int32),
            pltpu.VMEM((_CH, d), jnp.float32),
            pltpu.VMEM_SHARED((n, d), jnp.float32),
        ])


def _make_sc_deg(n, d, e):
    ew = e // _NW
    nchunk = ew // _CH
    nwin = nchunk // _WIN
    _NT = 10
    rpt = n // _NT

    mesh = plsc.VectorSubcoreMesh(core_axis_name="c", subcore_axis_name="s")

    def body(dst_hbm, z_hbm, ones_hbm, out_hbm, dst_v, ones_v, acc_sh):
        c = lax.axis_index("c")
        s = lax.axis_index("s")
        w = s * _NC + c
        base = s * rpt

        @pl.when(s < _NT)
        def _zero_acc():
            pltpu.sync_copy(z_hbm.at[pl.ds(base, rpt)],
                            acc_sh.at[pl.ds(base, rpt)])
        pltpu.sync_copy(ones_hbm, ones_v)

        plsc.subcore_barrier()

        def _win(wi, carry):
            pltpu.sync_copy(dst_hbm.at[w * nwin + wi], dst_v)

            def _edge(j, carry2):
                pltpu.sync_copy(ones_v, acc_sh.at[dst_v.at[j]], add=True)
                return carry2
            lax.fori_loop(0, _WIN, _edge, 0)
            return carry
        lax.fori_loop(0, nwin, _win, 0)

        plsc.subcore_barrier()

        @pl.when(s < _NT)
        def _readout():
            pltpu.sync_copy(acc_sh.at[pl.ds(base, rpt)],
                            out_hbm.at[pl.ds(c * n + base, rpt)])

    return pl.kernel(
        body, out_type=jax.ShapeDtypeStruct((_NC * n, d), jnp.float32),
        mesh=mesh,
        scratch_types=[
            pltpu.VMEM((_WIN, _CH), jnp.int32),
            pltpu.VMEM((_CH, d), jnp.float32),
            pltpu.VMEM_SHARED((n, d), jnp.float32),
        ])


def _lin_body(x_ref, w_ref, b_ref, o_ref):
    o_ref[...] = (jnp.dot(x_ref[...], w_ref[...],
                          preferred_element_type=jnp.float32) + b_ref[...])


def _combine_body(h_ref, p_ref, g_ref, o_ref, r_ref):
    deg = g_ref[0, :, 0:1] + g_ref[1, :, 0:1]
    rdeg = 1.0 / jnp.maximum(deg, 1.0)
    mean = (p_ref[0] + p_ref[1]) * rdeg
    o_ref[...] = jnp.maximum(0.5 * (h_ref[...] + mean), 0.0)
    r_ref[...] = jnp.broadcast_to(rdeg, h_ref.shape)


def _combine_mm_body(h_ref, p_ref, r_ref, w_ref, b_ref, o_ref):
    t = 0.5 * (h_ref[...] + (p_ref[0] + p_ref[1]) * r_ref[...])
    o_ref[...] = jnp.maximum(
        jnp.dot(t, w_ref[...], preferred_element_type=jnp.float32)
        + b_ref[...], 0.0)


def kernel(x, edge_index, W1, b1, W2, b2, W3, b3):
    n, d = x.shape
    e = edge_index.shape[1]
    ew = e // _NW
    nchunk = ew // _CH
    nwin = nchunk // _WIN
    src = edge_index[0].astype(jnp.int32).reshape(_NW * nwin, _WIN, _CH)
    dst = edge_index[1].astype(jnp.int32).reshape(_NW * nwin, _WIN, _CH)
    z_nd = jnp.zeros((n, d), jnp.float32)
    ones_chd = jnp.ones((_CH, d), jnp.float32)

    deg_k = _make_sc_deg(n, d, e)
    conv = _make_sc_conv(n, d, e)

    blk = 1000
    grid = (n // blk,)
    f32 = jnp.float32

    def _spec_h(i):
        return (i, 0)

    def _spec_w(i):
        return (0, 0)

    def _spec_p(i):
        return (0, i, 0)

    lin1 = pl.pallas_call(
        _lin_body, grid=grid,
        in_specs=[pl.BlockSpec((blk, d), _spec_h),
                  pl.BlockSpec((d, d), _spec_w),
                  pl.BlockSpec((1, d), _spec_w)],
        out_specs=pl.BlockSpec((blk, d), _spec_h),
        out_shape=jax.ShapeDtypeStruct((n, d), f32))

    combine = pl.pallas_call(
        _combine_body, grid=grid,
        in_specs=[pl.BlockSpec((blk, d), _spec_h),
                  pl.BlockSpec((_NC, blk, d), _spec_p),
                  pl.BlockSpec((_NC, blk, d), _spec_p)],
        out_specs=[pl.BlockSpec((blk, d), _spec_h),
                   pl.BlockSpec((blk, d), _spec_h)],
        out_shape=[jax.ShapeDtypeStruct((n, d), f32),
                   jax.ShapeDtypeStruct((n, d), f32)])

    combine_mm = pl.pallas_call(
        _combine_mm_body, grid=grid,
        in_specs=[pl.BlockSpec((blk, d), _spec_h),
                  pl.BlockSpec((_NC, blk, d), _spec_p),
                  pl.BlockSpec((blk, d), _spec_h),
                  pl.BlockSpec((d, d), _spec_w),
                  pl.BlockSpec((1, d), _spec_w)],
        out_specs=pl.BlockSpec((blk, d), _spec_h),
        out_shape=jax.ShapeDtypeStruct((n, d), f32))

    h0 = lin1(x, W1, b1.reshape(1, d))
    degp = deg_k(dst, z_nd, ones_chd)
    p1 = conv(h0, src, dst, z_nd)
    h1, rdeg = combine(h0, p1.reshape(_NC, n, d), degp.reshape(_NC, n, d))
    p2 = conv(h1, src, dst, z_nd)
    h2 = combine_mm(h1, p2.reshape(_NC, n, d), rdeg, W2, b2.reshape(1, d))
    p3 = conv(h2, src, dst, z_nd)
    h3 = combine_mm(h2, p3.reshape(_NC, n, d), rdeg, W3, b3.reshape(1, d))
    return h3
